# baseline (speedup 1.0000x reference)
import jax
import jax.numpy as jnp
from jax.experimental import pallas as pl
from jax.experimental.pallas import tpu as pltpu

_N = 2048
_NUM_PHYS = 600000
_THRESH = 1.0
_BLK = 256
_HALF = _N // 2
_W = _HALF + _BLK
_NSTRIP = _N // _BLK


def _notch_kernel(gate_ref, xc_ref, yc_ref, hxc_ref, hyc_ref,
                  xr_ref, yr_ref, hxr_ref, hyr_ref, out_ref,
                  w_ref, acc_ref):
    r = pl.program_id(0)

    @pl.when(r == 0)
    def _():
        lrow = jax.lax.broadcasted_iota(jnp.int32, (_BLK, _W), 0)
        lcol = jax.lax.broadcasted_iota(jnp.int32, (_BLK, _W), 1)
        t = lcol - lrow
        w = jnp.where((t >= 1) & (t < _HALF), 1.0,
                      jnp.where(t == _HALF, 0.5, 0.0))
        w_ref[...] = w.astype(jnp.float32)
        acc_ref[...] = jnp.zeros((_BLK, _W), jnp.float32)

    base = r * _BLK
    xc = xc_ref[...]
    yc = yc_ref[...]
    hxc = hxc_ref[...]
    hyc = hyc_ref[...]
    xr = xr_ref[:, pl.ds(base, _W)]
    yr = yr_ref[:, pl.ds(base, _W)]
    hxr = hxr_ref[:, pl.ds(base, _W)]
    hyr = hyr_ref[:, pl.ds(base, _W)]
    dx = jnp.maximum(jnp.abs(xc - xr) - (hxc + hxr), 0.0)
    dy = jnp.maximum(jnp.abs(yc - yr) - (hyc + hyr), 0.0)
    p = jnp.maximum((_THRESH - dx) - dy, 0.0)
    acc_ref[...] += w_ref[...] * (p * p)

    @pl.when(r == _NSTRIP - 1)
    def _():
        out_ref[0, 0] = jnp.sum(acc_ref[...]) * gate_ref[0, 0]


def kernel(pos, macro_mask, macro_size_x, macro_size_y):
    x = jax.lax.slice(pos, (0,), (_N,))
    y = jax.lax.slice(pos, (_NUM_PHYS,), (_NUM_PHYS + _N,))
    m = macro_mask
    neg = jnp.where(m, jnp.float32(0.0), jnp.float32(-1e7))
    hx = macro_size_x.astype(jnp.float32) * 0.5 + neg
    hy = macro_size_y.astype(jnp.float32) * 0.5 + neg
    count = jnp.sum(m.astype(jnp.int32))
    gate = jnp.where(count < 2, 0.0, 1.0).astype(jnp.float32).reshape(1, 1)

    col = lambda v: v.reshape(_N, 1)
    dbl = lambda v: jnp.concatenate([v, v]).reshape(1, 2 * _N)

    out = pl.pallas_call(
        _notch_kernel,
        grid=(_NSTRIP,),
        in_specs=[
            pl.BlockSpec(memory_space=pltpu.SMEM),
            pl.BlockSpec((_BLK, 1), lambda r: (r, 0)),
            pl.BlockSpec((_BLK, 1), lambda r: (r, 0)),
            pl.BlockSpec((_BLK, 1), lambda r: (r, 0)),
            pl.BlockSpec((_BLK, 1), lambda r: (r, 0)),
            pl.BlockSpec((1, 2 * _N), lambda r: (0, 0)),
            pl.BlockSpec((1, 2 * _N), lambda r: (0, 0)),
            pl.BlockSpec((1, 2 * _N), lambda r: (0, 0)),
            pl.BlockSpec((1, 2 * _N), lambda r: (0, 0)),
        ],
        out_shape=jax.ShapeDtypeStruct((1, 1), jnp.float32),
        out_specs=pl.BlockSpec(memory_space=pltpu.SMEM),
        scratch_shapes=[
            pltpu.VMEM((_BLK, _W), jnp.float32),
            pltpu.VMEM((_BLK, _W), jnp.float32),
        ],
        compiler_params=pltpu.CompilerParams(
            dimension_semantics=("arbitrary",)),
    )(gate, col(x), col(y), col(hx), col(hy), dbl(x), dbl(y), dbl(hx), dbl(hy))

    return out.reshape(())
